# Initial kernel scaffold; baseline (speedup 1.0000x reference)
import jax
import jax.numpy as jnp
from jax.experimental import pallas as pl


def kernel(x, CodeBook):
    raise NotImplementedError("write your pallas kernel here")


# speedup vs baseline: 1.0592x; 1.0592x over previous
import functools

import jax
import jax.numpy as jnp
from jax import lax
from jax.experimental import pallas as pl
from jax.experimental.pallas import tpu as pltpu
from jax.experimental.pallas import tpu_sc as plsc

N_TOK = 36864
N_CW = 8192
D = 64

BN = 256
KC = 2048
NKC = N_CW // KC

_NC = 2
_NS = 16
_NW = _NC * _NS
_BPW = N_TOK // _NW
_CH = 128
_NCH = _BPW // _CH


def _argmax_body(x_ref, x2_ref, cbt_ref, c2_ref, out_ref):
    x_blk = x_ref[...]

    def chunk(j, carry):
        run_min, run_arg = carry
        cbt_blk = cbt_ref[:, pl.ds(j * KC, KC)]
        mm = lax.dot_general(
            x_blk, cbt_blk, (((1,), (0,)), ((), ())),
            preferred_element_type=jnp.float32)
        d = (x2_ref[...] + c2_ref[:, pl.ds(j * KC, KC)]) - 2.0 * mm
        cm = jnp.min(d, axis=1, keepdims=True)
        iota = lax.broadcasted_iota(jnp.int32, (BN, KC), 1) + j * KC
        ci = jnp.min(jnp.where(d == cm, iota, N_CW),
                     axis=1, keepdims=True)
        better = cm < run_min
        return (jnp.where(better, cm, run_min),
                jnp.where(better, ci, run_arg))

    init = (jnp.full((BN, 1), jnp.inf, jnp.float32),
            jnp.zeros((BN, 1), jnp.int32))
    _, arg = lax.fori_loop(0, NKC, chunk, init)
    out_ref[...] = arg


_argmax_call = pl.pallas_call(
    _argmax_body,
    grid=(N_TOK // BN,),
    in_specs=[
        pl.BlockSpec((BN, D), lambda i: (i, 0)),
        pl.BlockSpec((BN, 1), lambda i: (i, 0)),
        pl.BlockSpec((D, N_CW), lambda i: (0, 0)),
        pl.BlockSpec((1, N_CW), lambda i: (0, 0)),
    ],
    out_specs=pl.BlockSpec((BN, 1), lambda i: (i, 0)),
    out_shape=jax.ShapeDtypeStruct((N_TOK, 1), jnp.int32),
)


def _gather_body(cb_hbm, idx_hbm, out_hbm, idx_v, rows_v, sem):
    wid = lax.axis_index("s") * _NC + lax.axis_index("c")
    pltpu.sync_copy(idx_hbm.at[pl.ds(wid * _BPW, _BPW)], idx_v)
    copies = [
        pltpu.async_copy(cb_hbm.at[idx_v.at[pl.ds(j * _CH, _CH)]],
                         rows_v.at[pl.ds(j * _CH, _CH)], sem)
        for j in range(_NCH)
    ]
    for cp in copies:
        cp.wait()
    pltpu.sync_copy(rows_v, out_hbm.at[pl.ds(wid * _BPW, _BPW)])


def _make_gather_call():
    return functools.partial(
        pl.kernel,
        out_type=jax.ShapeDtypeStruct((N_TOK, D), jnp.float32),
        scratch_types=[
            pltpu.VMEM((_BPW,), jnp.int32),
            pltpu.VMEM((_BPW, D), jnp.float32),
            pltpu.SemaphoreType.DMA,
        ],
        mesh=plsc.VectorSubcoreMesh(core_axis_name="c",
                                    subcore_axis_name="s"),
        compiler_params=pltpu.CompilerParams(use_tc_tiling_on_sc=False),
    )(_gather_body)


def kernel(x, CodeBook):
    x2 = jnp.sum(x * x, axis=1, keepdims=True)
    c2 = jnp.sum(CodeBook * CodeBook, axis=1)[None, :]
    cbt = CodeBook.T
    max_id = _argmax_call(x, x2, cbt, c2).reshape(N_TOK)
    q_hard = _make_gather_call()(CodeBook, max_id)
    return (q_hard, max_id)


# speedup vs baseline: 1.5412x; 1.4551x over previous
import functools

import jax
import jax.numpy as jnp
from jax import lax
from jax.experimental import pallas as pl
from jax.experimental.pallas import tpu as pltpu
from jax.experimental.pallas import tpu_sc as plsc

N_TOK = 36864
N_CW = 8192
D = 64

BN = 256
KC = 2048
NKC = N_CW // KC

_NC = 2
_NS = 16
_NW = _NC * _NS
_BPW = N_TOK // _NW
_CH = 128
_NCH = _BPW // _CH


def _argmax_body(x_ref, x2_ref, cbt_ref, c2_ref, out_ref):
    mm = lax.dot_general(
        x_ref[...], cbt_ref[...], (((1,), (0,)), ((), ())),
        preferred_element_type=jnp.float32)
    d = (x2_ref[...] + c2_ref[...]) - 2.0 * mm
    out_ref[...] = jnp.argmin(d, axis=1).astype(jnp.int32)[:, None]


_argmax_call = pl.pallas_call(
    _argmax_body,
    grid=(N_TOK // BN,),
    in_specs=[
        pl.BlockSpec((BN, D), lambda i: (i, 0)),
        pl.BlockSpec((BN, 1), lambda i: (i, 0)),
        pl.BlockSpec((D, N_CW), lambda i: (0, 0)),
        pl.BlockSpec((1, N_CW), lambda i: (0, 0)),
    ],
    out_specs=pl.BlockSpec((BN, 1), lambda i: (i, 0)),
    out_shape=jax.ShapeDtypeStruct((N_TOK, 1), jnp.int32),
)


def _gather_body(cb_hbm, idx_hbm, out_hbm, idx_v, rows_v, sem):
    wid = lax.axis_index("s") * _NC + lax.axis_index("c")
    pltpu.sync_copy(idx_hbm.at[pl.ds(wid * _BPW, _BPW)], idx_v)
    copies = [
        pltpu.async_copy(cb_hbm.at[idx_v.at[pl.ds(j * _CH, _CH)]],
                         rows_v.at[pl.ds(j * _CH, _CH)], sem)
        for j in range(_NCH)
    ]
    for cp in copies:
        cp.wait()
    pltpu.sync_copy(rows_v, out_hbm.at[pl.ds(wid * _BPW, _BPW)])


def _make_gather_call():
    return functools.partial(
        pl.kernel,
        out_type=jax.ShapeDtypeStruct((N_TOK, D), jnp.float32),
        scratch_types=[
            pltpu.VMEM((_BPW,), jnp.int32),
            pltpu.VMEM((_BPW, D), jnp.float32),
            pltpu.SemaphoreType.DMA,
        ],
        mesh=plsc.VectorSubcoreMesh(core_axis_name="c",
                                    subcore_axis_name="s"),
        compiler_params=pltpu.CompilerParams(use_tc_tiling_on_sc=False),
    )(_gather_body)


def kernel(x, CodeBook):
    x2 = jnp.sum(x * x, axis=1, keepdims=True)
    c2 = jnp.sum(CodeBook * CodeBook, axis=1)[None, :]
    cbt = CodeBook.T
    max_id = _argmax_call(x, x2, cbt, c2).reshape(N_TOK)
    q_hard = _make_gather_call()(CodeBook, max_id)
    return (q_hard, max_id)
